# baseline (speedup 1.0000x reference)
import functools

import jax
import jax.numpy as jnp
from jax import lax
from jax.experimental import pallas as pl
from jax.experimental.pallas import tpu as pltpu
from jax.experimental.pallas import tpu_sc as plsc

D_MODEL = 256
MAX_REL = 128
LENGTH = 512
V_ROWS = 2 * MAX_REL + 1
V_PAD = 264
PAD = LENGTH - MAX_REL - 1
C_ROWS = 1016
STAGE_ROW = 384
HALF = 128
NUM_CORES = 2
NUM_SUBCORES = 16
M_PER_TILE = 32
LANES = 16


def _sc_body(table_hbm, out_hbm, buf_v, sem):
    s = lax.axis_index("s")
    c = lax.axis_index("c")
    w = s * NUM_CORES + c
    r = lax.div(w, 4)
    rest = lax.rem(w, 4)
    d0 = lax.rem(rest, 2) * HALF
    m0 = lax.div(rest, 2) * M_PER_TILE

    pltpu.sync_copy(
        table_hbm.at[pl.ds(0, V_PAD), pl.ds(pl.multiple_of(d0, HALF), HALF)],
        buf_v.at[pl.ds(STAGE_ROW, V_PAD)],
    )

    lo = PAD - 7 + r
    hi = lo + V_ROWS

    def _move(k, carry):
        src = k + 8 - r
        for h in range(HALF // LANES):
            buf_v[k, pl.ds(h * LANES, LANES)] = (
                buf_v[src, pl.ds(h * LANES, LANES)]
            )
        return carry

    lax.fori_loop(lo, hi, _move, 0)

    bot = [buf_v[lo, pl.ds(h * LANES, LANES)] for h in range(HALF // LANES)]
    top = [
        buf_v[hi - 1, pl.ds(h * LANES, LANES)] for h in range(HALF // LANES)
    ]

    def _fill_bot_blk(b, carry):
        base = b * 8
        for j in range(8):
            for h in range(HALF // LANES):
                buf_v[base + j, pl.ds(h * LANES, LANES)] = bot[h]
        return carry

    def _fill_top_blk(b, carry):
        base = (STAGE_ROW + 256) + b * 8
        for j in range(8):
            for h in range(HALF // LANES):
                buf_v[base + j, pl.ds(h * LANES, LANES)] = top[h]
        return carry

    def _fill_bot(k, carry):
        for h in range(HALF // LANES):
            buf_v[k, pl.ds(h * LANES, LANES)] = bot[h]
        return carry

    def _fill_top(k, carry):
        for h in range(HALF // LANES):
            buf_v[k, pl.ds(h * LANES, LANES)] = top[h]
        return carry

    lax.fori_loop(0, (PAD - 7) // 8, _fill_bot_blk, 0)
    lax.fori_loop(PAD - 7, lo, _fill_bot, 0)
    lax.fori_loop(hi, STAGE_ROW + 256, _fill_top, 0)
    lax.fori_loop(0, (C_ROWS - STAGE_ROW - 256) // 8, _fill_top_blk, 0)

    copies = []
    for jj in range(M_PER_TILE):
        m = m0 + jj
        i = r + 8 * m
        start = pl.multiple_of(504 - 8 * m, 8)
        copies.append(
            pltpu.async_copy(
                buf_v.at[pl.ds(start, LENGTH)],
                out_hbm.at[
                    i,
                    pl.ds(0, LENGTH),
                    pl.ds(pl.multiple_of(d0, HALF), HALF),
                ],
                sem,
            )
        )
    for cp in copies:
        cp.wait()


@jax.jit
def _rel_pos_gather(table):
    mesh = plsc.VectorSubcoreMesh(
        core_axis_name="c",
        subcore_axis_name="s",
        num_cores=NUM_CORES,
        num_subcores=NUM_SUBCORES,
    )
    run = functools.partial(
        pl.kernel,
        out_type=jax.ShapeDtypeStruct((LENGTH, LENGTH, D_MODEL), jnp.float32),
        mesh=mesh,
        scratch_types=[
            pltpu.VMEM((C_ROWS, HALF), jnp.float32),
            pltpu.SemaphoreType.DMA,
        ],
        compiler_params=pltpu.CompilerParams(use_tc_tiling_on_sc=True),
    )(_sc_body)
    padded = jnp.pad(table, ((0, V_PAD - V_ROWS), (0, 0)))
    return run(padded)


def kernel(relative_embeddings, length):
    del length
    return _rel_pos_gather(relative_embeddings)


# speedup vs baseline: 1.0062x; 1.0062x over previous
import functools

import jax
import jax.numpy as jnp
from jax import lax
from jax.experimental import pallas as pl
from jax.experimental.pallas import tpu as pltpu
from jax.experimental.pallas import tpu_sc as plsc

D_MODEL = 256
MAX_REL = 128
LENGTH = 512
V_ROWS = 2 * MAX_REL + 1
V_PAD = 264
PAD = LENGTH - MAX_REL - 1
C_ROWS = 1016
STAGE_ROW = 384
HALF = 128
NUM_CORES = 2
NUM_SUBCORES = 16
M_PER_TILE = 32
LANES = 16


def _sc_body(table_hbm, out_hbm, buf_v, sem):
    s = lax.axis_index("s")
    c = lax.axis_index("c")
    w = s * NUM_CORES + c
    r = lax.div(w, 4)
    rest = lax.rem(w, 4)
    d0 = lax.rem(rest, 2) * HALF
    m0 = lax.div(rest, 2) * M_PER_TILE

    pltpu.sync_copy(
        table_hbm.at[pl.ds(0, V_PAD), pl.ds(pl.multiple_of(d0, HALF), HALF)],
        buf_v.at[pl.ds(STAGE_ROW, V_PAD)],
    )

    lo = PAD - 7 + r
    hi = lo + V_ROWS

    def _move(k, carry):
        src = k + 8 - r
        for h in range(HALF // LANES):
            buf_v[k, pl.ds(h * LANES, LANES)] = (
                buf_v[src, pl.ds(h * LANES, LANES)]
            )
        return carry

    lax.fori_loop(lo, hi, _move, 0)

    bot = [buf_v[lo, pl.ds(h * LANES, LANES)] for h in range(HALF // LANES)]
    top = [
        buf_v[hi - 1, pl.ds(h * LANES, LANES)] for h in range(HALF // LANES)
    ]

    def _fill_bot(k, carry):
        for h in range(HALF // LANES):
            buf_v[k, pl.ds(h * LANES, LANES)] = bot[h]
        return carry

    def _fill_top(k, carry):
        for h in range(HALF // LANES):
            buf_v[k, pl.ds(h * LANES, LANES)] = top[h]
        return carry

    lax.fori_loop(0, lo, _fill_bot, 0)
    lax.fori_loop(hi, C_ROWS, _fill_top, 0)

    copies = []
    for jj in range(M_PER_TILE):
        m = m0 + jj
        i = r + 8 * m
        start = pl.multiple_of(504 - 8 * m, 8)
        copies.append(
            pltpu.async_copy(
                buf_v.at[pl.ds(start, LENGTH)],
                out_hbm.at[
                    i,
                    pl.ds(0, LENGTH),
                    pl.ds(pl.multiple_of(d0, HALF), HALF),
                ],
                sem,
            )
        )
    for cp in copies:
        cp.wait()


@jax.jit
def _rel_pos_gather(table):
    mesh = plsc.VectorSubcoreMesh(
        core_axis_name="c",
        subcore_axis_name="s",
        num_cores=NUM_CORES,
        num_subcores=NUM_SUBCORES,
    )
    run = functools.partial(
        pl.kernel,
        out_type=jax.ShapeDtypeStruct((LENGTH, LENGTH, D_MODEL), jnp.float32),
        mesh=mesh,
        scratch_types=[
            pltpu.VMEM((C_ROWS, HALF), jnp.float32),
            pltpu.SemaphoreType.DMA,
        ],
        compiler_params=pltpu.CompilerParams(use_tc_tiling_on_sc=True),
    )(_sc_body)
    padded = jnp.pad(table, ((0, V_PAD - V_ROWS), (0, 0)))
    return run(padded)


def kernel(relative_embeddings, length):
    del length
    return _rel_pos_gather(relative_embeddings)


# speedup vs baseline: 1.0134x; 1.0071x over previous
import functools

import jax
import jax.numpy as jnp
from jax import lax
from jax.experimental import pallas as pl
from jax.experimental.pallas import tpu as pltpu
from jax.experimental.pallas import tpu_sc as plsc

D_MODEL = 256
MAX_REL = 128
LENGTH = 512
V_ROWS = 2 * MAX_REL + 1
V_PAD = 264
PAD = LENGTH - MAX_REL - 1
C_ROWS = 1016
STAGE_ROW = 384
HALF = 128
NUM_CORES = 2
NUM_SUBCORES = 16
M_PER_TILE = 32
LANES = 16


def _sc_body(table_hbm, out_hbm, buf_v, sem):
    s = lax.axis_index("s")
    c = lax.axis_index("c")
    w = s * NUM_CORES + c
    r = lax.div(w, 4)
    rest = lax.rem(w, 4)
    d0 = lax.rem(rest, 2) * HALF
    m0 = lax.div(rest, 2) * M_PER_TILE

    pltpu.sync_copy(
        table_hbm.at[pl.ds(0, V_PAD), pl.ds(pl.multiple_of(d0, HALF), HALF)],
        buf_v.at[pl.ds(STAGE_ROW, V_PAD)],
    )

    lo = PAD - 7 + r
    hi = lo + V_ROWS

    def _move(k, carry):
        src = k + 8 - r
        for h in range(HALF // LANES):
            buf_v[k, pl.ds(h * LANES, LANES)] = (
                buf_v[src, pl.ds(h * LANES, LANES)]
            )
        return carry

    lax.fori_loop(lo, hi, _move, 0)

    bot = [buf_v[lo, pl.ds(h * LANES, LANES)] for h in range(HALF // LANES)]
    top = [
        buf_v[hi - 1, pl.ds(h * LANES, LANES)] for h in range(HALF // LANES)
    ]

    def _fill_bot(k, carry):
        for h in range(HALF // LANES):
            buf_v[k, pl.ds(h * LANES, LANES)] = bot[h]
        return carry

    def _fill_top(k, carry):
        for h in range(HALF // LANES):
            buf_v[k, pl.ds(h * LANES, LANES)] = top[h]
        return carry

    lax.fori_loop(256 - 8 * m0, lo, _fill_bot, 0)
    lax.fori_loop(hi, C_ROWS - 8 * m0, _fill_top, 0)

    copies = []
    for jj in range(M_PER_TILE):
        m = m0 + jj
        i = r + 8 * m
        start = pl.multiple_of(504 - 8 * m, 8)
        copies.append(
            pltpu.async_copy(
                buf_v.at[pl.ds(start, LENGTH)],
                out_hbm.at[
                    i,
                    pl.ds(0, LENGTH),
                    pl.ds(pl.multiple_of(d0, HALF), HALF),
                ],
                sem,
            )
        )
    for cp in copies:
        cp.wait()


@jax.jit
def _rel_pos_gather(table):
    mesh = plsc.VectorSubcoreMesh(
        core_axis_name="c",
        subcore_axis_name="s",
        num_cores=NUM_CORES,
        num_subcores=NUM_SUBCORES,
    )
    run = functools.partial(
        pl.kernel,
        out_type=jax.ShapeDtypeStruct((LENGTH, LENGTH, D_MODEL), jnp.float32),
        mesh=mesh,
        scratch_types=[
            pltpu.VMEM((C_ROWS, HALF), jnp.float32),
            pltpu.SemaphoreType.DMA,
        ],
        compiler_params=pltpu.CompilerParams(use_tc_tiling_on_sc=True),
    )(_sc_body)
    padded = jnp.pad(table, ((0, V_PAD - V_ROWS), (0, 0)))
    return run(padded)


def kernel(relative_embeddings, length):
    del length
    return _rel_pos_gather(relative_embeddings)
